# baseline (speedup 1.0000x reference)
import jax
import jax.numpy as jnp
from jax import lax
from jax.experimental import pallas as pl
from jax.experimental.pallas import tpu as pltpu
from jax.experimental.pallas import tpu_sc as plsc

_B, _C, _H, _W = 8, 384, 14, 14
_E, _K, _R = 8, 2, 4
_HW = _H * _W
_T = _B * _HW
_HID = _R * _C
_CAP = int(1.25 * _T * _K / _E)
_CP = 512
_NSLOT = _E * _CP
_NW = 32
_SPW = _NSLOT // _NW
_PAIRS = 3328
_BIG = 1 << 30
_TPW = 56


def _stage1_body(x3_ref, dwk_ref, dwb_ref, lnw_ref, lnb_ref, rw_ref,
                 xn_ref, fd0_ref, fd1_ref, cc0_ref, cc1_ref,
                 gv0_ref, gv1_ref):
    x = x3_ref[...]
    rpos = lax.broadcasted_iota(jnp.int32, (_HW, 1), 0)
    wpos = rpos % _W
    hpos = rpos // _W
    acc = jnp.zeros((_B, _HW, _C), jnp.float32)
    for di in range(-3, 4):
        for dj in range(-3, 4):
            s = di * _W + dj
            tap = (di + 3) * 7 + (dj + 3)
            shifted = pltpu.roll(x, (-s) % _HW, axis=1)
            valid = ((wpos + dj >= 0) & (wpos + dj < _W)
                     & (hpos + di >= 0) & (hpos + di < _H))
            mk = jnp.where(valid, 1.0, 0.0) * dwk_ref[tap]
            acc = acc + shifted * mk
    y = acc + dwb_ref[...]

    mu = jnp.mean(y, axis=2, keepdims=True)
    var = jnp.mean((y - mu) * (y - mu), axis=2, keepdims=True)
    xn3 = (y - mu) * lax.rsqrt(var + 1e-6) * lnw_ref[...] + lnb_ref[...]
    xn_ref[...] = xn3

    e_iota = lax.broadcasted_iota(jnp.int32, (_HW, _E), 1)
    p1s, i1s, i2s, w1s, w2s, masks = [], [], [], [], [], []
    for b in range(_B):
        logits = jnp.dot(xn3[b], rw_ref[...], preferred_element_type=jnp.float32)
        m = jnp.max(logits, axis=1, keepdims=True)
        ex = jnp.exp(logits - m)
        probs = ex / jnp.sum(ex, axis=1, keepdims=True)
        p1 = jnp.max(probs, axis=1, keepdims=True)
        i1 = jnp.min(jnp.where(probs == p1, e_iota, _E), axis=1, keepdims=True)
        probs2 = jnp.where(e_iota == i1, -1.0, probs)
        p2 = jnp.max(probs2, axis=1, keepdims=True)
        i2 = jnp.min(jnp.where(probs2 == p2, e_iota, _E), axis=1, keepdims=True)
        denom = p1 + p2
        p1s.append(p1); i1s.append(i1); i2s.append(i2)
        w1s.append(p1 / denom); w2s.append(p2 / denom)
        masks.append(((e_iota == i1) | (e_iota == i2)).astype(jnp.float32))

    pT = jnp.concatenate(p1s, axis=1)
    p2d = jnp.transpose(pT)
    b_iota_col = lax.broadcasted_iota(jnp.int32, (_HW, _B), 1)
    b_iota_row = lax.broadcasted_iota(jnp.int32, (_B, _HW), 0)
    ri = lax.broadcasted_iota(jnp.int32, (_HW, _HW), 0)
    ci = lax.broadcasted_iota(jnp.int32, (_HW, _HW), 1)

    for br in range(_B):
        prow = jnp.sum(jnp.where(b_iota_col == br, pT, 0.0), axis=1,
                       keepdims=True)
        rank = jnp.zeros((_HW, _E), jnp.float32)
        for bc in range(_B):
            pcol = jnp.sum(jnp.where(b_iota_row == bc, p2d, 0.0), axis=0,
                           keepdims=True)
            gt = pcol > prow
            eq = pcol == prow
            if bc < br:
                cmpb = gt | eq
            elif bc > br:
                cmpb = gt
            else:
                cmpb = gt | (eq & (ci < ri))
            rank = rank + jnp.dot(cmpb.astype(jnp.float32), masks[bc],
                                  preferred_element_type=jnp.float32)
        r1 = jnp.sum(rank * (e_iota == i1s[br]), axis=1, keepdims=True)
        r2 = jnp.sum(rank * (e_iota == i2s[br]), axis=1, keepdims=True)
        k1 = r1 < _CAP
        k2 = r2 < _CAP
        f1 = i1s[br] * _CP + r1.astype(jnp.int32)
        f2 = i2s[br] * _CP + r2.astype(jnp.int32)
        fd0_ref[br] = jnp.where(k1, f1, _BIG)
        fd1_ref[br] = jnp.where(k2, f2, _BIG)
        cc0_ref[br] = jnp.where(k1, f1, _CP - 1)
        cc1_ref[br] = jnp.where(k2, f2, _CP - 1)
        gv0_ref[br] = jnp.where(k1, w1s[br], 0.0)
        gv1_ref[br] = jnp.where(k2, w2s[br], 0.0)


def _dispatch_body(xn_hbm, fdp_hbm, tokp_hbm, gvp_hbm, xd_hbm, sg_hbm,
                   fd_v, tok_v, gv_v, gidx_v, sgl_v, rows_v, sem):
    wid = lax.axis_index("s") * 2 + lax.axis_index("c")
    lo = wid * _SPW
    pltpu.sync_copy(fdp_hbm, fd_v)
    pltpu.sync_copy(tokp_hbm, tok_v)
    pltpu.sync_copy(gvp_hbm, gv_v)
    for j in range(_SPW // 16):
        gidx_v[pl.ds(j * 16, 16)] = jnp.zeros((16,), jnp.int32)
        sgl_v[pl.ds(j * 16, 16)] = jnp.zeros((16,), jnp.float32)

    def body(i, carry):
        f = fd_v[pl.ds(i * 16, 16)]
        t = tok_v[pl.ds(i * 16, 16)]
        g = gv_v[pl.ds(i * 16, 16)]
        loc = f - lo
        m = (f >= lo) & (f < lo + _SPW)
        plsc.store_scatter(gidx_v, [loc], t, mask=m)
        plsc.store_scatter(sgl_v, [loc], g, mask=m)
        return carry

    lax.fori_loop(0, _PAIRS // 16, body, 0)
    pltpu.async_copy(xn_hbm.at[gidx_v], rows_v, sem).wait()
    pltpu.sync_copy(rows_v, xd_hbm.at[pl.ds(lo, _SPW)])
    pltpu.sync_copy(sgl_v, sg_hbm.at[pl.ds(lo, _SPW)])


def _expert2_body(xd_ref, w1_ref, b1_ref, w2_ref, b2_ref, sg_ref, ls_ref,
                  ys_ref):
    x = xd_ref[0]
    h = jnp.dot(x, w1_ref[0], preferred_element_type=jnp.float32) + b1_ref[0]
    h = 0.5 * h * (1.0 + lax.erf(h * 0.7071067811865476))
    y = jnp.dot(h, w2_ref[0], preferred_element_type=jnp.float32) + b2_ref[0]
    ys_ref[0] = (sg_ref[0] * ls_ref[...]) * y


def _combine_body(x3_hbm, xn_hbm, ys_hbm, c0_hbm, c1_hbm, out_hbm,
                  c0_v, c1_v, xa_v, xb_v, y0_v, y1_v, sem):
    wid = lax.axis_index("s") * 2 + lax.axis_index("c")

    @pl.when(wid < _T // _TPW)
    def _():
        base = wid * _TPW
        pltpu.sync_copy(c0_hbm.at[pl.ds(base, _TPW)], c0_v)
        pltpu.sync_copy(c1_hbm.at[pl.ds(base, _TPW)], c1_v)
        pltpu.sync_copy(x3_hbm.at[pl.ds(base, _TPW)], xa_v)
        pltpu.sync_copy(xn_hbm.at[pl.ds(base, _TPW)], xb_v)
        pltpu.async_copy(ys_hbm.at[c0_v], y0_v, sem).wait()
        pltpu.async_copy(ys_hbm.at[c1_v], y1_v, sem).wait()

        def body(r, carry):
            for c in range(_C // 16):
                s = pl.ds(c * 16, 16)
                xa_v[r, s] = xa_v[r, s] + xb_v[r, s] + y0_v[r, s] + y1_v[r, s]
            return carry

        lax.fori_loop(0, _TPW, body, 0)
        pltpu.sync_copy(xa_v, out_hbm.at[pl.ds(base, _TPW)])


def kernel(input, dw_w, dw_b, ln_w, ln_b, router_w, w1, b1, w2, b2, layer_scale):
    x3 = jnp.transpose(input, (0, 2, 3, 1)).reshape(_B, _HW, _C)
    dwk = jnp.transpose(dw_w[:, 0], (1, 2, 0)).reshape(49, 1, _C)
    ls_row = layer_scale.reshape(1, _C)

    xn, fd0, fd1, cc0, cc1, gv0, gv1 = pl.pallas_call(
        _stage1_body,
        out_shape=[
            jax.ShapeDtypeStruct((_B, _HW, _C), jnp.float32),
            jax.ShapeDtypeStruct((_B, _HW, 1), jnp.int32),
            jax.ShapeDtypeStruct((_B, _HW, 1), jnp.int32),
            jax.ShapeDtypeStruct((_B, _HW, 1), jnp.int32),
            jax.ShapeDtypeStruct((_B, _HW, 1), jnp.int32),
            jax.ShapeDtypeStruct((_B, _HW, 1), jnp.float32),
            jax.ShapeDtypeStruct((_B, _HW, 1), jnp.float32),
        ],
    )(x3, dwk, dw_b, ln_w, ln_b, router_w)

    npad = _PAIRS - 2 * _T
    fdp = jnp.concatenate([fd0.reshape(-1), fd1.reshape(-1),
                           jnp.full((npad,), _BIG, jnp.int32)])
    tokp = jnp.concatenate([jnp.arange(_T, dtype=jnp.int32),
                            jnp.arange(_T, dtype=jnp.int32),
                            jnp.zeros((npad,), jnp.int32)])
    gvp = jnp.concatenate([gv0.reshape(-1), gv1.reshape(-1),
                           jnp.zeros((npad,), jnp.float32)])
    xnf = xn.reshape(_T, _C)
    x3f = x3.reshape(_T, _C)

    mesh = plsc.VectorSubcoreMesh(core_axis_name="c", subcore_axis_name="s")
    xd, sg = pl.kernel(
        _dispatch_body,
        mesh=mesh,
        compiler_params=pltpu.CompilerParams(needs_layout_passes=False),
        out_type=[
            jax.ShapeDtypeStruct((_NSLOT, _C), jnp.float32),
            jax.ShapeDtypeStruct((_NSLOT,), jnp.float32),
        ],
        scratch_types=[
            pltpu.VMEM((_PAIRS,), jnp.int32),
            pltpu.VMEM((_PAIRS,), jnp.int32),
            pltpu.VMEM((_PAIRS,), jnp.float32),
            pltpu.VMEM((_SPW,), jnp.int32),
            pltpu.VMEM((_SPW,), jnp.float32),
            pltpu.VMEM((_SPW, _C), jnp.float32),
            pltpu.SemaphoreType.DMA,
        ],
    )(xnf, fdp, tokp, gvp)

    ys = pl.pallas_call(
        _expert2_body,
        grid=(_E,),
        in_specs=[
            pl.BlockSpec((1, _CP, _C), lambda e: (e, 0, 0)),
            pl.BlockSpec((1, _C, _HID), lambda e: (e, 0, 0)),
            pl.BlockSpec((1, 1, _HID), lambda e: (e, 0, 0)),
            pl.BlockSpec((1, _HID, _C), lambda e: (e, 0, 0)),
            pl.BlockSpec((1, 1, _C), lambda e: (e, 0, 0)),
            pl.BlockSpec((1, _CP, 1), lambda e: (e, 0, 0)),
            pl.BlockSpec((1, _C), lambda e: (0, 0)),
        ],
        out_specs=pl.BlockSpec((1, _CP, _C), lambda e: (e, 0, 0)),
        out_shape=jax.ShapeDtypeStruct((_E, _CP, _C), jnp.float32),
    )(xd.reshape(_E, _CP, _C), w1, b1.reshape(_E, 1, _HID), w2,
      b2.reshape(_E, 1, _C), sg.reshape(_E, _CP, 1), ls_row)

    out = pl.kernel(
        _combine_body,
        mesh=mesh,
        out_type=jax.ShapeDtypeStruct((_T, _C), jnp.float32),
        scratch_types=[
            pltpu.VMEM((_TPW,), jnp.int32),
            pltpu.VMEM((_TPW,), jnp.int32),
            pltpu.VMEM((_TPW, _C), jnp.float32),
            pltpu.VMEM((_TPW, _C), jnp.float32),
            pltpu.VMEM((_TPW, _C), jnp.float32),
            pltpu.VMEM((_TPW, _C), jnp.float32),
            pltpu.SemaphoreType.DMA,
        ],
    )(x3f, xnf, ys.reshape(_NSLOT, _C), cc0.reshape(-1), cc1.reshape(-1))

    return jnp.transpose(out.reshape(_B, _H, _W, _C), (0, 3, 1, 2))


# speedup vs baseline: 1.0410x; 1.0410x over previous
import jax
import jax.numpy as jnp
from jax import lax
from jax.experimental import pallas as pl
from jax.experimental.pallas import tpu as pltpu
from jax.experimental.pallas import tpu_sc as plsc

_B, _C, _H, _W = 8, 384, 14, 14
_E, _K, _R = 8, 2, 4
_HW = _H * _W
_T = _B * _HW
_HID = _R * _C
_CAP = int(1.25 * _T * _K / _E)
_CP = 512
_NSLOT = _E * _CP
_NW = 32
_SPW = _NSLOT // _NW
_TPW = 56


def _stage1_body(x3_ref, dwk_ref, dwb_ref, lnw_ref, lnb_ref, rw_ref,
                 xn_ref, xs_ref, gidx_ref, sg_ref, cc0_ref, cc1_ref):
    x = x3_ref[...]
    rpos = lax.broadcasted_iota(jnp.int32, (_HW, 1), 0)
    wpos = rpos % _W
    hpos = rpos // _W
    acc = jnp.zeros((_B, _HW, _C), jnp.float32)
    for di in range(-3, 4):
        for dj in range(-3, 4):
            s = di * _W + dj
            tap = (di + 3) * 7 + (dj + 3)
            shifted = pltpu.roll(x, (-s) % _HW, axis=1)
            valid = ((wpos + dj >= 0) & (wpos + dj < _W)
                     & (hpos + di >= 0) & (hpos + di < _H))
            mk = jnp.where(valid, 1.0, 0.0) * dwk_ref[tap]
            acc = acc + shifted * mk
    y = acc + dwb_ref[...]

    mu = jnp.mean(y, axis=2, keepdims=True)
    var = jnp.mean((y - mu) * (y - mu), axis=2, keepdims=True)
    xn3 = (y - mu) * lax.rsqrt(var + 1e-6) * lnw_ref[...] + lnb_ref[...]
    xn_ref[...] = xn3
    xs_ref[...] = x + xn3

    e_iota = lax.broadcasted_iota(jnp.int32, (_HW, _E), 1)
    p1s, i1s, i2s, w1s, w2s, masks = [], [], [], [], [], []
    for b in range(_B):
        logits = jnp.dot(xn3[b], rw_ref[...], preferred_element_type=jnp.float32)
        m = jnp.max(logits, axis=1, keepdims=True)
        ex = jnp.exp(logits - m)
        probs = ex / jnp.sum(ex, axis=1, keepdims=True)
        p1 = jnp.max(probs, axis=1, keepdims=True)
        i1 = jnp.min(jnp.where(probs == p1, e_iota, _E), axis=1, keepdims=True)
        probs2 = jnp.where(e_iota == i1, -1.0, probs)
        p2 = jnp.max(probs2, axis=1, keepdims=True)
        i2 = jnp.min(jnp.where(probs2 == p2, e_iota, _E), axis=1, keepdims=True)
        denom = p1 + p2
        p1s.append(p1); i1s.append(i1); i2s.append(i2)
        w1s.append(p1 / denom); w2s.append(p2 / denom)
        masks.append(((e_iota == i1) | (e_iota == i2)).astype(jnp.float32))

    pT = jnp.concatenate(p1s, axis=1)
    p2d = jnp.transpose(pT)
    b_iota_col = lax.broadcasted_iota(jnp.int32, (_HW, _B), 1)
    b_iota_row = lax.broadcasted_iota(jnp.int32, (_B, _HW), 0)
    ri = lax.broadcasted_iota(jnp.int32, (_HW, _HW), 0)
    ci = lax.broadcasted_iota(jnp.int32, (_HW, _HW), 1)
    be_iota = lax.broadcasted_iota(jnp.int32, (_E, _HW), 0)
    s_iota = lax.broadcasted_iota(jnp.int32, (_CP, 1), 0).astype(jnp.float32)
    tok0 = lax.broadcasted_iota(jnp.int32, (_HW, 1), 0).astype(jnp.float32)

    gacc = [jnp.zeros((_CP, 1), jnp.float32) for _ in range(_E)]
    sacc = [jnp.zeros((_CP, 1), jnp.float32) for _ in range(_E)]

    for br in range(_B):
        prow = jnp.sum(jnp.where(b_iota_col == br, pT, 0.0), axis=1,
                       keepdims=True)
        rank = jnp.zeros((_HW, _E), jnp.float32)
        for bc in range(_B):
            pcol = jnp.sum(jnp.where(b_iota_row == bc, p2d, 0.0), axis=0,
                           keepdims=True)
            gt = pcol > prow
            eq = pcol == prow
            if bc < br:
                cmpb = gt | eq
            elif bc > br:
                cmpb = gt
            else:
                cmpb = gt | (eq & (ci < ri))
            rank = rank + jnp.dot(cmpb.astype(jnp.float32), masks[bc],
                                  preferred_element_type=jnp.float32)
        r1 = jnp.sum(rank * (e_iota == i1s[br]), axis=1, keepdims=True)
        r2 = jnp.sum(rank * (e_iota == i2s[br]), axis=1, keepdims=True)
        k1 = r1 < _CAP
        k2 = r2 < _CAP
        f1 = i1s[br] * _CP + r1.astype(jnp.int32)
        f2 = i2s[br] * _CP + r2.astype(jnp.int32)
        cc0_ref[br] = jnp.where(k1, f1, _CP - 1)
        cc1_ref[br] = jnp.where(k2, f2, _CP - 1)
        g1 = jnp.where(k1, w1s[br], 0.0)
        g2 = jnp.where(k2, w2s[br], 0.0)
        gates = g1 * (e_iota == i1s[br]) + g2 * (e_iota == i2s[br])

        routed_kept = (masks[br] > 0.0) & (rank < _CAP)
        se = jnp.where(routed_kept, rank, -1.0)
        seT = jnp.transpose(se)
        tok = tok0 + (br * _HW)
        for e in range(_E):
            se_row = jnp.sum(jnp.where(be_iota == e, seT, 0.0), axis=0,
                             keepdims=True)
            onehot = (s_iota == se_row).astype(jnp.float32)
            ge_col = jnp.sum(gates * (e_iota == e), axis=1, keepdims=True)
            gacc[e] = gacc[e] + jnp.dot(onehot, tok,
                                        preferred_element_type=jnp.float32)
            sacc[e] = sacc[e] + jnp.dot(onehot, ge_col,
                                        preferred_element_type=jnp.float32)

    for e in range(_E):
        gidx_ref[e] = gacc[e].astype(jnp.int32)
        sg_ref[e] = sacc[e]


def _dispatch_body(xn_hbm, gidx_hbm, xd_hbm, gidx_v, rows_v, sem):
    wid = lax.axis_index("s") * 2 + lax.axis_index("c")
    lo = wid * _SPW
    pltpu.sync_copy(gidx_hbm.at[pl.ds(lo, _SPW)], gidx_v)
    pltpu.async_copy(xn_hbm.at[gidx_v], rows_v, sem).wait()
    pltpu.sync_copy(rows_v, xd_hbm.at[pl.ds(lo, _SPW)])


def _expert2_body(xd_ref, w1_ref, b1_ref, w2_ref, b2_ref, sg_ref, ls_ref,
                  ys_ref):
    x = xd_ref[0]
    h = jnp.dot(x, w1_ref[0], preferred_element_type=jnp.float32) + b1_ref[0]
    h = 0.5 * h * (1.0 + lax.erf(h * 0.7071067811865476))
    y = jnp.dot(h, w2_ref[0], preferred_element_type=jnp.float32) + b2_ref[0]
    ys_ref[0] = (sg_ref[0] * ls_ref[...]) * y


def _combine_body(xs_hbm, ys_hbm, c0_hbm, c1_hbm, out_hbm,
                  c0_v, c1_v, xa_v, y0_v, y1_v, sem):
    wid = lax.axis_index("s") * 2 + lax.axis_index("c")

    @pl.when(wid < _T // _TPW)
    def _():
        base = wid * _TPW
        pltpu.sync_copy(c0_hbm.at[pl.ds(base, _TPW)], c0_v)
        pltpu.sync_copy(c1_hbm.at[pl.ds(base, _TPW)], c1_v)
        pltpu.sync_copy(xs_hbm.at[pl.ds(base, _TPW)], xa_v)
        cp0 = pltpu.async_copy(ys_hbm.at[c0_v], y0_v, sem)
        cp1 = pltpu.async_copy(ys_hbm.at[c1_v], y1_v, sem)
        cp0.wait()
        cp1.wait()

        def body(r, carry):
            for c in range(_C // 16):
                s = pl.ds(c * 16, 16)
                xa_v[r, s] = xa_v[r, s] + y0_v[r, s] + y1_v[r, s]
            return carry

        lax.fori_loop(0, _TPW, body, 0)
        pltpu.sync_copy(xa_v, out_hbm.at[pl.ds(base, _TPW)])


def kernel(input, dw_w, dw_b, ln_w, ln_b, router_w, w1, b1, w2, b2, layer_scale):
    x3 = jnp.transpose(input, (0, 2, 3, 1)).reshape(_B, _HW, _C)
    dwk = jnp.transpose(dw_w[:, 0], (1, 2, 0)).reshape(49, 1, _C)
    ls_row = layer_scale.reshape(1, _C)

    xn, xs, gidx, sg, cc0, cc1 = pl.pallas_call(
        _stage1_body,
        out_shape=[
            jax.ShapeDtypeStruct((_B, _HW, _C), jnp.float32),
            jax.ShapeDtypeStruct((_B, _HW, _C), jnp.float32),
            jax.ShapeDtypeStruct((_E, _CP, 1), jnp.int32),
            jax.ShapeDtypeStruct((_E, _CP, 1), jnp.float32),
            jax.ShapeDtypeStruct((_B, _HW, 1), jnp.int32),
            jax.ShapeDtypeStruct((_B, _HW, 1), jnp.int32),
        ],
    )(x3, dwk, dw_b, ln_w, ln_b, router_w)

    xnf = xn.reshape(_T, _C)
    xsf = xs.reshape(_T, _C)

    mesh = plsc.VectorSubcoreMesh(core_axis_name="c", subcore_axis_name="s")
    xd = pl.kernel(
        _dispatch_body,
        mesh=mesh,
        out_type=jax.ShapeDtypeStruct((_NSLOT, _C), jnp.float32),
        scratch_types=[
            pltpu.VMEM((_SPW,), jnp.int32),
            pltpu.VMEM((_SPW, _C), jnp.float32),
            pltpu.SemaphoreType.DMA,
        ],
    )(xnf, gidx.reshape(-1))

    ys = pl.pallas_call(
        _expert2_body,
        grid=(_E,),
        in_specs=[
            pl.BlockSpec((1, _CP, _C), lambda e: (e, 0, 0)),
            pl.BlockSpec((1, _C, _HID), lambda e: (e, 0, 0)),
            pl.BlockSpec((1, 1, _HID), lambda e: (e, 0, 0)),
            pl.BlockSpec((1, _HID, _C), lambda e: (e, 0, 0)),
            pl.BlockSpec((1, 1, _C), lambda e: (e, 0, 0)),
            pl.BlockSpec((1, _CP, 1), lambda e: (e, 0, 0)),
            pl.BlockSpec((1, _C), lambda e: (0, 0)),
        ],
        out_specs=pl.BlockSpec((1, _CP, _C), lambda e: (e, 0, 0)),
        out_shape=jax.ShapeDtypeStruct((_E, _CP, _C), jnp.float32),
    )(xd.reshape(_E, _CP, _C), w1, b1.reshape(_E, 1, _HID), w2,
      b2.reshape(_E, 1, _C), sg, ls_row)

    out = pl.kernel(
        _combine_body,
        mesh=mesh,
        out_type=jax.ShapeDtypeStruct((_T, _C), jnp.float32),
        scratch_types=[
            pltpu.VMEM((_TPW,), jnp.int32),
            pltpu.VMEM((_TPW,), jnp.int32),
            pltpu.VMEM((_TPW, _C), jnp.float32),
            pltpu.VMEM((_TPW, _C), jnp.float32),
            pltpu.VMEM((_TPW, _C), jnp.float32),
            pltpu.SemaphoreType.DMA,
        ],
    )(xsf, ys.reshape(_NSLOT, _C), cc0.reshape(-1), cc1.reshape(-1))

    return jnp.transpose(out.reshape(_B, _H, _W, _C), (0, 3, 1, 2))


# speedup vs baseline: 1.4967x; 1.4377x over previous
import jax
import jax.numpy as jnp
from jax import lax
from jax.experimental import pallas as pl
from jax.experimental.pallas import tpu as pltpu
from jax.experimental.pallas import tpu_sc as plsc

_B, _C, _H, _W = 8, 384, 14, 14
_E, _K, _R = 8, 2, 4
_HW = _H * _W
_T = _B * _HW
_HID = _R * _C
_CAP = int(1.25 * _T * _K / _E)
_CP = 512
_NSLOT = _E * _CP
_NW = 32
_SPW = _NSLOT // _NW
_TPW = 56


def _stage1_body(x3_ref, dwk_ref, dwb_ref, lnw_ref, lnb_ref, rw_ref,
                 xs_ref, xd_ref, sg_ref, cc0_ref, cc1_ref):
    x = x3_ref[...]
    rpos = lax.broadcasted_iota(jnp.int32, (_HW, 1), 0)
    wpos = rpos % _W
    hpos = rpos // _W
    acc = jnp.zeros((_B, _HW, _C), jnp.float32)
    for di in range(-3, 4):
        for dj in range(-3, 4):
            s = di * _W + dj
            tap = (di + 3) * 7 + (dj + 3)
            shifted = pltpu.roll(x, (-s) % _HW, axis=1)
            valid = ((wpos + dj >= 0) & (wpos + dj < _W)
                     & (hpos + di >= 0) & (hpos + di < _H))
            mk = jnp.where(valid, 1.0, 0.0) * dwk_ref[tap]
            acc = acc + shifted * mk
    y = acc + dwb_ref[...]

    mu = jnp.mean(y, axis=2, keepdims=True)
    var = jnp.mean((y - mu) * (y - mu), axis=2, keepdims=True)
    xn3 = (y - mu) * lax.rsqrt(var + 1e-6) * lnw_ref[...] + lnb_ref[...]
    xs_ref[...] = x + xn3

    e_iota = lax.broadcasted_iota(jnp.int32, (_HW, _E), 1)
    p1s, i1s, i2s, w1s, w2s, masks = [], [], [], [], [], []
    for b in range(_B):
        logits = jnp.dot(xn3[b], rw_ref[...], preferred_element_type=jnp.float32)
        m = jnp.max(logits, axis=1, keepdims=True)
        ex = jnp.exp(logits - m)
        probs = ex / jnp.sum(ex, axis=1, keepdims=True)
        p1 = jnp.max(probs, axis=1, keepdims=True)
        i1 = jnp.min(jnp.where(probs == p1, e_iota, _E), axis=1, keepdims=True)
        probs2 = jnp.where(e_iota == i1, -1.0, probs)
        p2 = jnp.max(probs2, axis=1, keepdims=True)
        i2 = jnp.min(jnp.where(probs2 == p2, e_iota, _E), axis=1, keepdims=True)
        denom = p1 + p2
        p1s.append(p1); i1s.append(i1); i2s.append(i2)
        w1s.append(p1 / denom); w2s.append(p2 / denom)
        masks.append(((e_iota == i1) | (e_iota == i2)).astype(jnp.float32))

    pT = jnp.concatenate(p1s, axis=1)
    p2d = jnp.transpose(pT)
    b_iota_col = lax.broadcasted_iota(jnp.int32, (_HW, _B), 1)
    b_iota_row = lax.broadcasted_iota(jnp.int32, (_B, _HW), 0)
    ri = lax.broadcasted_iota(jnp.int32, (_HW, _HW), 0)
    ci = lax.broadcasted_iota(jnp.int32, (_HW, _HW), 1)
    be_iota = lax.broadcasted_iota(jnp.int32, (_E, _HW), 0)
    s_iota = lax.broadcasted_iota(jnp.int32, (_CP, 1), 0).astype(jnp.float32)

    xacc = [jnp.zeros((_CP, _C + 1), jnp.float32) for _ in range(_E)]

    for br in range(_B):
        prow = jnp.sum(jnp.where(b_iota_col == br, pT, 0.0), axis=1,
                       keepdims=True)
        rank = jnp.zeros((_HW, _E), jnp.float32)
        for bc in range(_B):
            pcol = jnp.sum(jnp.where(b_iota_row == bc, p2d, 0.0), axis=0,
                           keepdims=True)
            gt = pcol > prow
            eq = pcol == prow
            if bc < br:
                cmpb = gt | eq
            elif bc > br:
                cmpb = gt
            else:
                cmpb = gt | (eq & (ci < ri))
            rank = rank + jnp.dot(cmpb.astype(jnp.float32), masks[bc],
                                  preferred_element_type=jnp.float32)
        r1 = jnp.sum(rank * (e_iota == i1s[br]), axis=1, keepdims=True)
        r2 = jnp.sum(rank * (e_iota == i2s[br]), axis=1, keepdims=True)
        k1 = r1 < _CAP
        k2 = r2 < _CAP
        f1 = i1s[br] * _CP + r1.astype(jnp.int32)
        f2 = i2s[br] * _CP + r2.astype(jnp.int32)
        cc0_ref[br] = jnp.where(k1, f1, _CP - 1)
        cc1_ref[br] = jnp.where(k2, f2, _CP - 1)
        g1 = jnp.where(k1, w1s[br], 0.0)
        g2 = jnp.where(k2, w2s[br], 0.0)
        gates = g1 * (e_iota == i1s[br]) + g2 * (e_iota == i2s[br])

        routed_kept = (masks[br] > 0.0) & (rank < _CAP)
        se = jnp.where(routed_kept, rank, -1.0)
        seT = jnp.transpose(se)
        for e in range(_E):
            se_row = jnp.sum(jnp.where(be_iota == e, seT, 0.0), axis=0,
                             keepdims=True)
            onehot = (s_iota == se_row).astype(jnp.float32)
            ge_col = jnp.sum(gates * (e_iota == e), axis=1, keepdims=True)
            rhs = jnp.concatenate([xn3[br], ge_col], axis=1)
            xacc[e] = xacc[e] + jnp.dot(onehot, rhs,
                                        preferred_element_type=jnp.float32)

    for e in range(_E):
        xd_ref[e] = xacc[e][:, :_C]
        sg_ref[e] = xacc[e][:, _C:_C + 1]


def _expert2_body(xd_ref, w1_ref, b1_ref, w2_ref, b2_ref, sg_ref, ls_ref,
                  ys_ref):
    x = xd_ref[0]
    h = jnp.dot(x, w1_ref[0], preferred_element_type=jnp.float32) + b1_ref[0]
    h = 0.5 * h * (1.0 + lax.erf(h * 0.7071067811865476))
    y = jnp.dot(h, w2_ref[0], preferred_element_type=jnp.float32) + b2_ref[0]
    ys_ref[0] = (sg_ref[0] * ls_ref[...]) * y


def _combine_body(xs_hbm, ys_hbm, c0_hbm, c1_hbm, out_hbm,
                  c0_v, c1_v, xa_v, y0_v, y1_v, sem):
    wid = lax.axis_index("s") * 2 + lax.axis_index("c")

    @pl.when(wid < _T // _TPW)
    def _():
        base = wid * _TPW
        pltpu.sync_copy(c0_hbm.at[pl.ds(base, _TPW)], c0_v)
        pltpu.sync_copy(c1_hbm.at[pl.ds(base, _TPW)], c1_v)
        pltpu.sync_copy(xs_hbm.at[pl.ds(base, _TPW)], xa_v)
        cp0 = pltpu.async_copy(ys_hbm.at[c0_v], y0_v, sem)
        cp1 = pltpu.async_copy(ys_hbm.at[c1_v], y1_v, sem)
        cp0.wait()
        cp1.wait()

        def body(r, carry):
            for c in range(_C // 16):
                s = pl.ds(c * 16, 16)
                xa_v[r, s] = xa_v[r, s] + y0_v[r, s] + y1_v[r, s]
            return carry

        lax.fori_loop(0, _TPW, body, 0)
        pltpu.sync_copy(xa_v, out_hbm.at[pl.ds(base, _TPW)])


def kernel(input, dw_w, dw_b, ln_w, ln_b, router_w, w1, b1, w2, b2, layer_scale):
    x3 = jnp.transpose(input, (0, 2, 3, 1)).reshape(_B, _HW, _C)
    dwk = jnp.transpose(dw_w[:, 0], (1, 2, 0)).reshape(49, 1, _C)
    ls_row = layer_scale.reshape(1, _C)

    xs, xd, sg, cc0, cc1 = pl.pallas_call(
        _stage1_body,
        out_shape=[
            jax.ShapeDtypeStruct((_B, _HW, _C), jnp.float32),
            jax.ShapeDtypeStruct((_E, _CP, _C), jnp.float32),
            jax.ShapeDtypeStruct((_E, _CP, 1), jnp.float32),
            jax.ShapeDtypeStruct((_B, _HW, 1), jnp.int32),
            jax.ShapeDtypeStruct((_B, _HW, 1), jnp.int32),
        ],
    )(x3, dwk, dw_b, ln_w, ln_b, router_w)

    xsf = xs.reshape(_T, _C)

    mesh = plsc.VectorSubcoreMesh(core_axis_name="c", subcore_axis_name="s")

    ys = pl.pallas_call(
        _expert2_body,
        grid=(_E,),
        in_specs=[
            pl.BlockSpec((1, _CP, _C), lambda e: (e, 0, 0)),
            pl.BlockSpec((1, _C, _HID), lambda e: (e, 0, 0)),
            pl.BlockSpec((1, 1, _HID), lambda e: (e, 0, 0)),
            pl.BlockSpec((1, _HID, _C), lambda e: (e, 0, 0)),
            pl.BlockSpec((1, 1, _C), lambda e: (e, 0, 0)),
            pl.BlockSpec((1, _CP, 1), lambda e: (e, 0, 0)),
            pl.BlockSpec((1, _C), lambda e: (0, 0)),
        ],
        out_specs=pl.BlockSpec((1, _CP, _C), lambda e: (e, 0, 0)),
        out_shape=jax.ShapeDtypeStruct((_E, _CP, _C), jnp.float32),
    )(xd, w1, b1.reshape(_E, 1, _HID), w2,
      b2.reshape(_E, 1, _C), sg, ls_row)

    out = pl.kernel(
        _combine_body,
        mesh=mesh,
        out_type=jax.ShapeDtypeStruct((_T, _C), jnp.float32),
        scratch_types=[
            pltpu.VMEM((_TPW,), jnp.int32),
            pltpu.VMEM((_TPW,), jnp.int32),
            pltpu.VMEM((_TPW, _C), jnp.float32),
            pltpu.VMEM((_TPW, _C), jnp.float32),
            pltpu.VMEM((_TPW, _C), jnp.float32),
            pltpu.SemaphoreType.DMA,
        ],
    )(xsf, ys.reshape(_NSLOT, _C), cc0.reshape(-1), cc1.reshape(-1))

    return jnp.transpose(out.reshape(_B, _H, _W, _C), (0, 3, 1, 2))
